# baseline (speedup 1.0000x reference)
import functools

import jax
import jax.numpy as jnp
from jax import lax
from jax.experimental import pallas as pl
from jax.experimental.pallas import tpu as pltpu
from jax.experimental.pallas import tpu_sc as plsc

NC = 2
NS = 16
NW = NC * NS
LANES = 16

CHUNK = 512
GATHER = 128
NBUF = 4


def _build(qp: int, n: int):
    n_chunks = qp // (NW * CHUNK)
    assert n_chunks % NBUF == 0
    mesh = plsc.VectorSubcoreMesh(core_axis_name="c", subcore_axis_name="s")

    @functools.partial(
        pl.kernel,
        mesh=mesh,
        out_type=jax.ShapeDtypeStruct((qp,), jnp.float32),
        scratch_types=(
            [pltpu.VMEM((CHUNK,), jnp.int32)] * (2 * NBUF)
            + [pltpu.VMEM((CHUNK,), jnp.float32)] * (3 * NBUF)
            + [pltpu.SemaphoreType.DMA] * (2 * NBUF)
        ),
    )
    def sc_kernel(x_hbm, y_hbm, ii_hbm, jj_hbm, out_hbm, *scr):
        wid = lax.axis_index("s") * NC + lax.axis_index("c")
        ii = scr[0:NBUF]
        jj = scr[NBUF:2 * NBUF]
        gx = scr[2 * NBUF:3 * NBUF]
        gy = scr[3 * NBUF:4 * NBUF]
        ov = scr[4 * NBUF:5 * NBUF]
        gsem = scr[5 * NBUF:6 * NBUF]
        ssem = scr[6 * NBUF:7 * NBUF]

        def stage_and_fire(ci, b):
            base = (wid * n_chunks + ci) * CHUNK
            pltpu.sync_copy(ii_hbm.at[pl.ds(base, CHUNK)], ii[b])
            pltpu.sync_copy(jj_hbm.at[pl.ds(base, CHUNK)], jj[b])
            for t in range(CHUNK // GATHER):
                sl = pl.ds(t * GATHER, GATHER)
                pltpu.async_copy(
                    x_hbm.at[ii[b].at[sl]], gx[b].at[sl], gsem[b])
                pltpu.async_copy(
                    y_hbm.at[jj[b].at[sl]], gy[b].at[sl], gsem[b])

        def drain_gathers(b):
            pltpu.make_async_copy(
                x_hbm.at[pl.ds(0, CHUNK)], gx[b], gsem[b]).wait()
            pltpu.make_async_copy(
                y_hbm.at[pl.ds(0, CHUNK)], gy[b], gsem[b]).wait()

        def drain_store(b):
            pltpu.make_async_copy(
                ov[b], out_hbm.at[pl.ds(0, CHUNK)], ssem[b]).wait()

        for b in range(NBUF):
            stage_and_fire(b, b)

        def outer(c0, _):
            for b in range(NBUF):
                ci = c0 * NBUF + b
                drain_gathers(b)

                @pl.when(ci >= NBUF)
                def _():
                    drain_store(b)

                def mul_body(k, _):
                    s = pl.ds(k * LANES, LANES)
                    ov[b][s] = gx[b][s] * gy[b][s]
                    return ()

                lax.fori_loop(0, CHUNK // LANES, mul_body, ())
                base = (wid * n_chunks + ci) * CHUNK
                pltpu.async_copy(ov[b], out_hbm.at[pl.ds(base, CHUNK)],
                                 ssem[b])

                @pl.when(ci + NBUF < n_chunks)
                def _():
                    stage_and_fire(ci + NBUF, b)

            return ()

        lax.fori_loop(0, n_chunks // NBUF, outer, ())
        for b in range(NBUF):
            drain_store(b)

    return sc_kernel


def kernel(x, y, idx_i, idx_j):
    q = idx_i.shape[0]
    step = NW * CHUNK * NBUF
    qp = ((q + step - 1) // step) * step
    pad = qp - q
    if pad:
        zeros = jnp.zeros((pad,), jnp.int32)
        ii = jnp.concatenate([idx_i, zeros])
        jj = jnp.concatenate([idx_j, zeros])
    else:
        ii, jj = idx_i, idx_j
    out = _build(qp, x.shape[0])(x, y, ii, jj)
    return out[:q]


# speedup vs baseline: 1.4163x; 1.4163x over previous
import functools

import jax
import jax.numpy as jnp
from jax import lax
from jax.experimental import pallas as pl
from jax.experimental.pallas import tpu as pltpu
from jax.experimental.pallas import tpu_sc as plsc

NC = 2
NS = 16
NW = NC * NS
LANES = 16

CHUNK = 512
GATHER = 128
NBUF = 7


def _build(qp: int, n: int):
    n_chunks = qp // (NW * CHUNK)
    assert n_chunks % NBUF == 0
    mesh = plsc.VectorSubcoreMesh(core_axis_name="c", subcore_axis_name="s")

    @functools.partial(
        pl.kernel,
        mesh=mesh,
        out_type=jax.ShapeDtypeStruct((qp,), jnp.float32),
        scratch_types=(
            [pltpu.VMEM((CHUNK,), jnp.int32)] * (2 * NBUF)
            + [pltpu.VMEM((CHUNK,), jnp.float32)] * (3 * NBUF)
            + [pltpu.SemaphoreType.DMA] * (2 * NBUF)
        ),
    )
    def sc_kernel(x_hbm, y_hbm, ii_hbm, jj_hbm, out_hbm, *scr):
        wid = lax.axis_index("s") * NC + lax.axis_index("c")
        ii = scr[0:NBUF]
        jj = scr[NBUF:2 * NBUF]
        gx = scr[2 * NBUF:3 * NBUF]
        gy = scr[3 * NBUF:4 * NBUF]
        ov = scr[4 * NBUF:5 * NBUF]
        gsem = scr[5 * NBUF:6 * NBUF]
        ssem = scr[6 * NBUF:7 * NBUF]

        def stage_and_fire(ci, b):
            base = (wid * n_chunks + ci) * CHUNK
            pltpu.sync_copy(ii_hbm.at[pl.ds(base, CHUNK)], ii[b])
            pltpu.sync_copy(jj_hbm.at[pl.ds(base, CHUNK)], jj[b])
            for t in range(CHUNK // GATHER):
                sl = pl.ds(t * GATHER, GATHER)
                pltpu.async_copy(
                    x_hbm.at[ii[b].at[sl]], gx[b].at[sl], gsem[b])
                pltpu.async_copy(
                    y_hbm.at[jj[b].at[sl]], gy[b].at[sl], gsem[b])

        def drain_gathers(b):
            pltpu.make_async_copy(
                x_hbm.at[pl.ds(0, CHUNK)], gx[b], gsem[b]).wait()
            pltpu.make_async_copy(
                y_hbm.at[pl.ds(0, CHUNK)], gy[b], gsem[b]).wait()

        def drain_store(b):
            pltpu.make_async_copy(
                ov[b], out_hbm.at[pl.ds(0, CHUNK)], ssem[b]).wait()

        for b in range(NBUF):
            stage_and_fire(b, b)

        def outer(c0, _):
            for b in range(NBUF):
                ci = c0 * NBUF + b
                drain_gathers(b)

                @pl.when(ci >= NBUF)
                def _():
                    drain_store(b)

                def mul_body(k, _):
                    s = pl.ds(k * LANES, LANES)
                    ov[b][s] = gx[b][s] * gy[b][s]
                    return ()

                lax.fori_loop(0, CHUNK // LANES, mul_body, ())
                base = (wid * n_chunks + ci) * CHUNK
                pltpu.async_copy(ov[b], out_hbm.at[pl.ds(base, CHUNK)],
                                 ssem[b])

                @pl.when(ci + NBUF < n_chunks)
                def _():
                    stage_and_fire(ci + NBUF, b)

            return ()

        lax.fori_loop(0, n_chunks // NBUF, outer, ())
        for b in range(NBUF):
            drain_store(b)

    return sc_kernel


def kernel(x, y, idx_i, idx_j):
    q = idx_i.shape[0]
    step = NW * CHUNK * NBUF
    qp = ((q + step - 1) // step) * step
    pad = qp - q
    if pad:
        zeros = jnp.zeros((pad,), jnp.int32)
        ii = jnp.concatenate([idx_i, zeros])
        jj = jnp.concatenate([idx_j, zeros])
    else:
        ii, jj = idx_i, idx_j
    out = _build(qp, x.shape[0])(x, y, ii, jj)
    return out[:q]


# speedup vs baseline: 1.6522x; 1.1666x over previous
import functools

import jax
import jax.numpy as jnp
from jax import lax
from jax.experimental import pallas as pl
from jax.experimental.pallas import tpu as pltpu
from jax.experimental.pallas import tpu_sc as plsc

NC = 2
NS = 16
NW = NC * NS
LANES = 16

CHUNK = 512
GATHER = 128
NBUF = 5


def _mul_offsets(n):
    offs = list(range(0, max(n - LANES, 0) + 1, LANES))
    if offs[-1] + LANES < n:
        offs.append(n - LANES)
    return offs


def _build(q: int):
    per_w = q // NW
    n_full = per_w // CHUNK
    tail = per_w - n_full * CHUNK
    assert q % NW == 0 and per_w % 8 == 0
    assert n_full >= NBUF and tail % 8 == 0 and (tail == 0 or tail >= LANES)
    mesh = plsc.VectorSubcoreMesh(core_axis_name="c", subcore_axis_name="s")

    @functools.partial(
        pl.kernel,
        mesh=mesh,
        out_type=jax.ShapeDtypeStruct((q,), jnp.float32),
        scratch_types=(
            [pltpu.VMEM((CHUNK,), jnp.int32)] * (2 * NBUF)
            + [pltpu.VMEM((CHUNK,), jnp.float32)] * (3 * NBUF)
            + [pltpu.SemaphoreType.DMA] * (2 * NBUF)
        ),
    )
    def sc_kernel(x_hbm, y_hbm, ii_hbm, jj_hbm, out_hbm, *scr):
        wid = lax.axis_index("s") * NC + lax.axis_index("c")
        base0 = wid * per_w
        ii = scr[0:NBUF]
        jj = scr[NBUF:2 * NBUF]
        gx = scr[2 * NBUF:3 * NBUF]
        gy = scr[3 * NBUF:4 * NBUF]
        ov = scr[4 * NBUF:5 * NBUF]
        gsem = scr[5 * NBUF:6 * NBUF]
        ssem = scr[6 * NBUF:7 * NBUF]

        def stage_and_fire(ci, b):
            base = base0 + ci * CHUNK
            pltpu.sync_copy(ii_hbm.at[pl.ds(base, CHUNK)], ii[b])
            pltpu.sync_copy(jj_hbm.at[pl.ds(base, CHUNK)], jj[b])
            for t in range(CHUNK // GATHER):
                sl = pl.ds(t * GATHER, GATHER)
                pltpu.async_copy(
                    x_hbm.at[ii[b].at[sl]], gx[b].at[sl], gsem[b])
                pltpu.async_copy(
                    y_hbm.at[jj[b].at[sl]], gy[b].at[sl], gsem[b])

        def drain_gathers(b):
            pltpu.make_async_copy(
                x_hbm.at[pl.ds(0, CHUNK)], gx[b], gsem[b]).wait()
            pltpu.make_async_copy(
                y_hbm.at[pl.ds(0, CHUNK)], gy[b], gsem[b]).wait()

        def drain_store(b):
            pltpu.make_async_copy(
                ov[b], out_hbm.at[pl.ds(0, CHUNK)], ssem[b]).wait()

        def process(ci, b, first_round):
            drain_gathers(b)
            if not first_round:
                drain_store(b)

            def mul_body(k, _):
                s = pl.ds(k * LANES, LANES)
                ov[b][s] = gx[b][s] * gy[b][s]
                return ()

            lax.fori_loop(0, CHUNK // LANES, mul_body, ())
            pltpu.async_copy(
                ov[b], out_hbm.at[pl.ds(base0 + ci * CHUNK, CHUNK)], ssem[b])

            @pl.when(ci + NBUF < n_full)
            def _():
                stage_and_fire(ci + NBUF, b)

        for b in range(NBUF):
            stage_and_fire(b, b)

        def outer_first(c0, _):
            for b in range(NBUF):
                process(c0 * NBUF + b, b, first_round=True)
            return ()

        def outer(c0, _):
            for b in range(NBUF):
                process(c0 * NBUF + b, b, first_round=False)
            return ()

        n_rounds = n_full // NBUF
        rem = n_full - n_rounds * NBUF
        lax.fori_loop(0, 1, outer_first, ())
        lax.fori_loop(1, n_rounds, outer, ())
        for b in range(rem):
            process(n_rounds * NBUF + b, b, first_round=False)

        for b in range(NBUF):
            drain_store(b)

        if tail:
            tb = base0 + n_full * CHUNK
            ts = pl.ds(0, tail)
            pltpu.sync_copy(ii_hbm.at[pl.ds(tb, tail)], ii[0].at[ts])
            pltpu.sync_copy(jj_hbm.at[pl.ds(tb, tail)], jj[0].at[ts])
            pltpu.async_copy(x_hbm.at[ii[0].at[ts]], gx[0].at[ts], gsem[0])
            pltpu.async_copy(y_hbm.at[jj[0].at[ts]], gy[0].at[ts], gsem[0])
            pltpu.make_async_copy(
                x_hbm.at[pl.ds(0, tail)], gx[0].at[ts], gsem[0]).wait()
            pltpu.make_async_copy(
                y_hbm.at[pl.ds(0, tail)], gy[0].at[ts], gsem[0]).wait()
            for o in _mul_offsets(tail):
                s = pl.ds(o, LANES)
                ov[0][s] = gx[0][s] * gy[0][s]
            pltpu.sync_copy(ov[0].at[ts], out_hbm.at[pl.ds(tb, tail)])

    return sc_kernel


def kernel(x, y, idx_i, idx_j):
    return _build(idx_i.shape[0])(x, y, idx_i, idx_j)
